# baseline (speedup 1.0000x reference)
import functools

import jax
import jax.numpy as jnp
from jax import lax
from jax.experimental import pallas as pl
from jax.experimental.pallas import tpu as pltpu
from jax.experimental.pallas import tpu_sc as plsc


def _sc_embedding_lookup(pos_table, np_vec):
    V, D = pos_table.shape
    info = plsc.get_sparse_core_info()
    NC, NS, L = 1, info.num_subcores, info.num_lanes
    NW = NC * NS
    b_per_w = V // NW
    mesh = plsc.VectorSubcoreMesh(
        core_axis_name="c", subcore_axis_name="s",
        num_cores=NC, num_subcores=NS,
    )

    @functools.partial(
        pl.kernel,
        mesh=mesh,
        out_type=jax.ShapeDtypeStruct((V, D), pos_table.dtype),
        scratch_types=[
            pltpu.VMEM((b_per_w,), jnp.int32),
            pltpu.VMEM((b_per_w, D), pos_table.dtype),
            pltpu.SemaphoreType.DMA,
        ],
    )
    def lookup_k(table_hbm, np_hbm, out_hbm, idx_v, rows_v, sem):
        wid = lax.axis_index("s") * NC + lax.axis_index("c")
        base = wid * b_per_w
        pltpu.sync_copy(np_hbm, idx_v.at[pl.ds(0, L)])
        nm1 = idx_v[pl.ds(0, L)] - 1
        for j in range(b_per_w // L):
            pos = lax.iota(jnp.int32, L) + (base + j * L)
            idx_v[pl.ds(j * L, L)] = jnp.minimum(pos, nm1)
        pltpu.async_copy(table_hbm.at[idx_v], rows_v, sem).wait()
        pltpu.sync_copy(rows_v, out_hbm.at[pl.ds(base, b_per_w)])

    return lookup_k(pos_table, np_vec)


def _add_select_kernel(np_ref, x_ref, table_ref, o_ref):
    n = np_ref[0]
    table = table_ref[...]
    last = table_ref[pl.ds(n - 1, 1), :]
    rows = jax.lax.broadcasted_iota(jnp.int32, (table.shape[0], 1), 0)
    enc = jnp.where(rows < n, table, last)
    o_ref[...] = x_ref[...] + enc[None]


def _add_enc_kernel(alias_ref, x_ref, enc_ref, o_ref):
    del alias_ref
    o_ref[...] = x_ref[...] + enc_ref[...][None]


def kernel(projected_patches, num_patches, pos_table):
    B, P, D = projected_patches.shape
    np_arr = jnp.asarray(num_patches, jnp.int32).reshape((1,))
    np_vec = jnp.full((16,), jnp.asarray(num_patches, jnp.int32), jnp.int32)
    encoded = _sc_embedding_lookup(pos_table, np_vec)

    BB = 4
    SPLIT = 48
    n1 = SPLIT // BB
    n2 = (B - SPLIT) // BB

    out1 = pl.pallas_call(
        _add_select_kernel,
        grid_spec=pltpu.PrefetchScalarGridSpec(
            num_scalar_prefetch=1,
            grid=(n1,),
            in_specs=[
                pl.BlockSpec((BB, P, D), lambda b, np_: (b, 0, 0)),
                pl.BlockSpec((P, D), lambda b, np_: (0, 0)),
            ],
            out_specs=pl.BlockSpec((BB, P, D), lambda b, np_: (b, 0, 0)),
        ),
        out_shape=jax.ShapeDtypeStruct((B, P, D), projected_patches.dtype),
    )(np_arr, projected_patches, pos_table)

    off = n1
    return pl.pallas_call(
        _add_enc_kernel,
        grid=(n2,),
        in_specs=[
            pl.BlockSpec(memory_space=pl.ANY),
            pl.BlockSpec((BB, P, D), lambda b: (b + off, 0, 0)),
            pl.BlockSpec((P, D), lambda b: (0, 0)),
        ],
        out_specs=pl.BlockSpec((BB, P, D), lambda b: (b + off, 0, 0)),
        out_shape=jax.ShapeDtypeStruct((B, P, D), projected_patches.dtype),
        input_output_aliases={0: 0},
    )(out1, projected_patches, encoded)


# speedup vs baseline: 1.0035x; 1.0035x over previous
import functools

import jax
import jax.numpy as jnp
from jax import lax
from jax.experimental import pallas as pl
from jax.experimental.pallas import tpu as pltpu
from jax.experimental.pallas import tpu_sc as plsc


def _sc_embedding_lookup(pos_table, np_vec):
    V, D = pos_table.shape
    info = plsc.get_sparse_core_info()
    NC, NS, L = 1, info.num_subcores, info.num_lanes
    NW = NC * NS
    b_per_w = V // NW
    mesh = plsc.VectorSubcoreMesh(
        core_axis_name="c", subcore_axis_name="s",
        num_cores=NC, num_subcores=NS,
    )

    @functools.partial(
        pl.kernel,
        mesh=mesh,
        out_type=jax.ShapeDtypeStruct((V, D), pos_table.dtype),
        scratch_types=[
            pltpu.VMEM((b_per_w,), jnp.int32),
            pltpu.VMEM((b_per_w, D), pos_table.dtype),
            pltpu.SemaphoreType.DMA,
        ],
    )
    def lookup_k(table_hbm, np_hbm, out_hbm, idx_v, rows_v, sem):
        wid = lax.axis_index("s") * NC + lax.axis_index("c")
        base = wid * b_per_w
        pltpu.sync_copy(np_hbm, idx_v.at[pl.ds(0, L)])
        nm1 = idx_v[pl.ds(0, L)] - 1
        for j in range(b_per_w // L):
            pos = lax.iota(jnp.int32, L) + (base + j * L)
            idx_v[pl.ds(j * L, L)] = jnp.minimum(pos, nm1)
        pltpu.async_copy(table_hbm.at[idx_v], rows_v, sem).wait()
        pltpu.sync_copy(rows_v, out_hbm.at[pl.ds(base, b_per_w)])

    return lookup_k(pos_table, np_vec)


def _add_select_kernel(np_ref, x_ref, table_ref, o_ref):
    n = np_ref[0]
    table = table_ref[...]
    last = table_ref[pl.ds(n - 1, 1), :]
    rows = jax.lax.broadcasted_iota(jnp.int32, (table.shape[0], 1), 0)
    enc = jnp.where(rows < n, table, last)
    o_ref[...] = x_ref[...] + enc[None]


def _add_enc_kernel(alias_ref, x_ref, enc_ref, o_ref):
    del alias_ref
    o_ref[...] = x_ref[...] + enc_ref[...][None]


def kernel(projected_patches, num_patches, pos_table):
    B, P, D = projected_patches.shape
    np_arr = jnp.asarray(num_patches, jnp.int32).reshape((1,))
    np_vec = jnp.full((16,), jnp.asarray(num_patches, jnp.int32), jnp.int32)
    encoded = _sc_embedding_lookup(pos_table, np_vec)

    BB = 4
    SPLIT = 56
    n1 = SPLIT // BB
    n2 = (B - SPLIT) // BB

    out1 = pl.pallas_call(
        _add_select_kernel,
        grid_spec=pltpu.PrefetchScalarGridSpec(
            num_scalar_prefetch=1,
            grid=(n1,),
            in_specs=[
                pl.BlockSpec((BB, P, D), lambda b, np_: (b, 0, 0)),
                pl.BlockSpec((P, D), lambda b, np_: (0, 0)),
            ],
            out_specs=pl.BlockSpec((BB, P, D), lambda b, np_: (b, 0, 0)),
        ),
        out_shape=jax.ShapeDtypeStruct((B, P, D), projected_patches.dtype),
    )(np_arr, projected_patches, pos_table)

    off = n1
    return pl.pallas_call(
        _add_enc_kernel,
        grid=(n2,),
        in_specs=[
            pl.BlockSpec(memory_space=pl.ANY),
            pl.BlockSpec((BB, P, D), lambda b: (b + off, 0, 0)),
            pl.BlockSpec((P, D), lambda b: (0, 0)),
        ],
        out_specs=pl.BlockSpec((BB, P, D), lambda b: (b + off, 0, 0)),
        out_shape=jax.ShapeDtypeStruct((B, P, D), projected_patches.dtype),
        input_output_aliases={0: 0},
    )(out1, projected_patches, encoded)
